# baseline (speedup 1.0000x reference)
import functools
import math

import jax
import jax.numpy as jnp
from jax import lax
from jax.experimental import pallas as pl
from jax.experimental.pallas import tpu as pltpu
from jax.experimental.pallas import tpu_sc as plsc

_D = 512
_G = 1000
_P = 8
_M = 16
_TOPK = 5
_ATHR = math.sin(1000.0 / (2.0 * 6371.0)) ** 2

_NC = 2
_NS = 16
_NW = _NC * _NS
_LANES = 16


def _pick_chunk(rpw, d):
    best = 8
    for c in range(8, 129, 8):
        if rpw % c == 0 and c * d * 4 <= 163840:
            best = c
    return best


def _gather_rows(table, idx, *, unroll=10):
    n, = idx.shape
    _, d = table.shape
    rpw = n // _NW
    assert rpw * _NW == n and rpw % 8 == 0
    chunk = _pick_chunk(rpw, d)
    nchunks = rpw // chunk
    mesh = plsc.VectorSubcoreMesh(core_axis_name="c", subcore_axis_name="s")

    @functools.partial(
        pl.kernel,
        mesh=mesh,
        compiler_params=pltpu.CompilerParams(use_tc_tiling_on_sc=(d % 128 == 0)),
        out_type=jax.ShapeDtypeStruct((n, d), table.dtype),
        scratch_types=[
            pltpu.VMEM((rpw,), jnp.int32),
            pltpu.VMEM((chunk, d), table.dtype),
            pltpu.VMEM((chunk, d), table.dtype),
            pltpu.SemaphoreType.DMA,
            pltpu.SemaphoreType.DMA,
            pltpu.SemaphoreType.DMA,
            pltpu.SemaphoreType.DMA,
        ],
    )
    def k(table_hbm, idx_hbm, out_hbm, idx_v, buf0, buf1,
          gsem0, gsem1, ssem0, ssem1):
        wid = lax.axis_index("s") * _NC + lax.axis_index("c")
        base = wid * rpw
        pltpu.sync_copy(idx_hbm.at[pl.ds(base, rpw)], idx_v)
        bufs = (buf0, buf1)
        gsems = (gsem0, gsem1)
        ssems = (ssem0, ssem1)

        def run_group(g, nin):
            gcp = {}
            scp = {}

            def start_gather(j):
                ci = g * unroll + j
                gcp[j] = pltpu.async_copy(
                    table_hbm.at[idx_v.at[pl.ds(ci * chunk, chunk)]],
                    bufs[j % 2],
                    gsems[j % 2],
                )

            start_gather(0)
            for j in range(nin):
                if j + 1 < nin:
                    if j - 1 >= 0:
                        scp[j - 1].wait()
                    start_gather(j + 1)
                gcp[j].wait()
                ci = g * unroll + j
                scp[j] = pltpu.async_copy(
                    bufs[j % 2],
                    out_hbm.at[pl.ds(base + ci * chunk, chunk)],
                    ssems[j % 2],
                )
            if nin - 2 >= 0:
                scp[nin - 2].wait()
            scp[nin - 1].wait()

        full, rem = divmod(nchunks, unroll)
        if full > 1:
            def body(g, c):
                run_group(g, unroll)
                return c
            lax.fori_loop(0, full, body, 0)
        elif full == 1:
            run_group(0, unroll)
        if rem:
            run_group(full, rem)

    return k(table, idx)


def _gather_mean16(table, idx, *, unroll=10):
    n, = idx.shape
    _, d = table.shape
    nprotos = n // _M
    ppw = nprotos // _NW
    rpw = n // _NW
    cp = 4
    chunk = cp * _M
    ngroups = ppw // (2 * cp)
    nlc = d // _LANES
    mesh = plsc.VectorSubcoreMesh(core_axis_name="c", subcore_axis_name="s")

    @functools.partial(
        pl.kernel,
        mesh=mesh,
        out_type=jax.ShapeDtypeStruct((nprotos, d), jnp.float32),
        scratch_types=[
            pltpu.VMEM((rpw,), jnp.int32),
            pltpu.VMEM((chunk, d), jnp.float32),
            pltpu.VMEM((chunk, d), jnp.float32),
            pltpu.VMEM((2 * cp, d), jnp.float32),
            pltpu.VMEM((2 * cp, d), jnp.float32),
            pltpu.SemaphoreType.DMA,
            pltpu.SemaphoreType.DMA,
            pltpu.SemaphoreType.DMA,
            pltpu.SemaphoreType.DMA,
        ],
    )
    def k(table_hbm, idx_hbm, out_hbm, idx_v, buf0, buf1, mb0, mb1,
          gsem0, gsem1, msem0, msem1):
        wid = lax.axis_index("s") * _NC + lax.axis_index("c")
        base = wid * rpw
        pbase = wid * ppw
        pltpu.sync_copy(idx_hbm.at[pl.ds(base, rpw)], idx_v)
        bufs = (buf0, buf1)
        mbufs = (mb0, mb1)
        gsems = (gsem0, gsem1)
        msems = (msem0, msem1)

        def start_gather(ci, side):
            return pltpu.async_copy(
                table_hbm.at[idx_v.at[pl.ds(ci * chunk, chunk)]],
                bufs[side],
                gsems[side],
            )

        def wait_gather(side):
            pltpu.make_async_copy(
                table_hbm.at[idx_v.at[pl.ds(0, chunk)]],
                bufs[side],
                gsems[side],
            ).wait()

        def wait_mstore(side):
            pltpu.make_async_copy(
                mbufs[side],
                out_hbm.at[pl.ds(pbase, 2 * cp)],
                msems[side],
            ).wait()

        def reduce_chunk(side, mb, mrow0):
            buf = bufs[side]

            def cbody(c, carry):
                col = pl.ds(c * _LANES, _LANES)
                for j in range(cp):
                    r = j * _M
                    v = [buf[r + m, col] for m in range(_M)]
                    t8 = [v[m] + v[m + 8] for m in range(8)]
                    t4 = [t8[m] + t8[m + 4] for m in range(4)]
                    t2 = [t4[m] + t4[m + 2] for m in range(2)]
                    ssum = t2[0] + t2[1]
                    mb[mrow0 + j, col] = ssum * jnp.float32(1.0 / _M)
                return carry
            lax.fori_loop(0, nlc, cbody, 0)

        start_gather(0, 0)
        start_gather(1, 1)

        def gbody(g, carry):
            ms = lax.rem(g, 2)
            mb = mb0
            @pl.when(g >= 2)
            def _():
                @pl.when(ms == 0)
                def _():
                    wait_mstore(0)
                @pl.when(ms == 1)
                def _():
                    wait_mstore(1)

            def half(side, mrow0, ci_next):
                wait_gather(side)
                @pl.when(ms == 0)
                def _():
                    reduce_chunk(side, mb0, mrow0)
                @pl.when(ms == 1)
                def _():
                    reduce_chunk(side, mb1, mrow0)
                @pl.when(ci_next < 2 * ngroups)
                def _():
                    start_gather(ci_next, side)

            half(0, 0, 2 * g + 2)
            half(1, cp, 2 * g + 3)

            @pl.when(ms == 0)
            def _():
                pltpu.async_copy(
                    mb0, out_hbm.at[pl.ds(pbase + g * 2 * cp, 2 * cp)], msem0)
            @pl.when(ms == 1)
            def _():
                pltpu.async_copy(
                    mb1, out_hbm.at[pl.ds(pbase + g * 2 * cp, 2 * cp)], msem1)
            return carry

        lax.fori_loop(0, ngroups, gbody, 0)
        if ngroups >= 2:
            wait_mstore(0)
        wait_mstore(1)

    return k(table, idx)


def _repack_members(pi3):
    def kfn(i_ref, o_ref):
        arr = i_ref[...]
        for p in range(_P):
            o_ref[:, p * _M:(p + 1) * _M] = arr[:, p, :]

    return pl.pallas_call(
        kfn,
        out_shape=jax.ShapeDtypeStruct((_G, _P * _M), jnp.int32),
    )(pi3)


def _proto_argmin(rows2, emb, cm):
    b = emb.shape[0]
    bb = 64

    def kfn(r_ref, e_ref, cm_ref, bd_ref, bm_ref):
        r = r_ref[...]
        e = e_ref[...]
        dd = r - e[:, None, None, :]
        s = jnp.sqrt(jnp.sum(dd * dd, axis=-1) + 1e-12)
        best = s[:, :, 0]
        bi = jnp.zeros(best.shape, jnp.int32)
        for p in range(1, _P):
            c = s[:, :, p]
            lt = c < best
            bi = jnp.where(lt, p, bi)
            best = jnp.where(lt, c, best)
        cmv = cm_ref[...].reshape(bb, _TOPK, _P * _M)
        bm = cmv[:, :, 0:_M]
        for p in range(1, _P):
            sel = (bi == p)[:, :, None]
            bm = jnp.where(sel, cmv[:, :, p * _M:(p + 1) * _M], bm)
        bd_ref[...] = best
        bm_ref[...] = bm

    return pl.pallas_call(
        kfn,
        grid=(b // bb,),
        in_specs=[
            pl.BlockSpec((bb, _TOPK, _P, _D), lambda i: (i, 0, 0, 0)),
            pl.BlockSpec((bb, _D), lambda i: (i, 0)),
            pl.BlockSpec((bb * _TOPK, _P * _M), lambda i: (i, 0)),
        ],
        out_specs=[
            pl.BlockSpec((bb, _TOPK), lambda i: (i, 0)),
            pl.BlockSpec((bb, _TOPK, _M), lambda i: (i, 0, 0)),
        ],
        out_shape=[
            jax.ShapeDtypeStruct((b, _TOPK), jnp.float32),
            jax.ShapeDtypeStruct((b, _TOPK, _M), jnp.int32),
        ],
    )(rows2, emb, cm)


def _member_argmin(rows3, emb, bm):
    b = emb.shape[0]
    bb = 64

    def kfn(r_ref, e_ref, bm_ref, bg_ref):
        r = r_ref[...]
        e = e_ref[...]
        dd = r - e[:, None, None, :]
        s = jnp.sqrt(jnp.sum(dd * dd, axis=-1) + 1e-12)
        best = s[:, :, 0]
        bi = jnp.zeros(best.shape, jnp.int32)
        for m in range(1, _M):
            c = s[:, :, m]
            lt = c < best
            bi = jnp.where(lt, m, bi)
            best = jnp.where(lt, c, best)
        bmv = bm_ref[...]
        bg = bmv[:, :, 0]
        for m in range(1, _M):
            bg = jnp.where(bi == m, bmv[:, :, m], bg)
        bg_ref[...] = bg

    return pl.pallas_call(
        kfn,
        grid=(b // bb,),
        in_specs=[
            pl.BlockSpec((bb, _TOPK, _M, _D), lambda i: (i, 0, 0, 0)),
            pl.BlockSpec((bb, _D), lambda i: (i, 0)),
            pl.BlockSpec((bb, _TOPK, _M), lambda i: (i, 0, 0)),
        ],
        out_specs=pl.BlockSpec((bb, _TOPK), lambda i: (i, 0)),
        out_shape=jax.ShapeDtypeStruct((b, _TOPK), jnp.int32),
    )(rows3, emb, bm)


def _finish(bd, cp5, cand5, latk, lngk, ipreds, temp):
    b = bd.shape[0]

    def kfn(bd_ref, cp_ref, cd_ref, la_ref, lo_ref, ip_ref, t_ref,
            olat_ref, olng_ref, ogc_ref):
        scores = -bd_ref[...]
        t = t_ref[0, 0]
        ex = jnp.exp(scores / t)
        probs = ex / jnp.sum(ex, axis=-1, keepdims=True)
        fp = cp_ref[...] * probs
        best = fp[:, 0]
        bi = jnp.zeros(best.shape, jnp.int32)
        for k2 in range(1, _TOPK):
            c = fp[:, k2]
            gt = c > best
            bi = jnp.where(gt, k2, bi)
            best = jnp.where(gt, c, best)
        la = la_ref[...]
        lo = lo_ref[...]
        rlat = la[:, 0]
        rlng = lo[:, 0]
        for k2 in range(1, _TOPK):
            rlat = jnp.where(bi == k2, la[:, k2], rlat)
            rlng = jnp.where(bi == k2, lo[:, k2], rlng)
        deg = jnp.float32(math.pi / 180.0)
        lat1 = ip_ref[:, 0] * deg
        lng1 = ip_ref[:, 1] * deg
        lat2 = rlat * deg
        lng2 = rlng * deg
        sdlat = jnp.sin((lat2 - lat1) * 0.5)
        sdlng = jnp.sin((lng2 - lng1) * 0.5)
        a = sdlat * sdlat + jnp.cos(lat1) * jnp.cos(lat2) * sdlng * sdlng
        fi = jnp.where(a > jnp.float32(_ATHR), 0, bi)
        flat = la[:, 0]
        flng = lo[:, 0]
        cd = cd_ref[...]
        gc = cd[:, 0]
        for k2 in range(1, _TOPK):
            sel = fi == k2
            flat = jnp.where(sel, la[:, k2], flat)
            flng = jnp.where(sel, lo[:, k2], flng)
            gc = jnp.where(sel, cd[:, k2], gc)
        olat_ref[...] = flat
        olng_ref[...] = flng
        ogc_ref[...] = gc

    return pl.pallas_call(
        kfn,
        out_shape=[
            jax.ShapeDtypeStruct((b,), jnp.float32),
            jax.ShapeDtypeStruct((b,), jnp.float32),
            jax.ShapeDtypeStruct((b,), cand5.dtype),
        ],
    )(bd, cp5, cand5, latk, lngk, ipreds, temp)


def kernel(embedding, initial_preds, candidate_cells, candidate_probs,
           embeddings, proto_indices, dataset_latlng, temperature):
    b, d = embedding.shape
    fm = _repack_members(proto_indices.astype(jnp.int32))
    idx1 = jnp.pad(fm.reshape(-1), (0, 256 * _NW * _M - _G * _P * _M))
    pm = _gather_mean16(embeddings, idx1)
    cand = candidate_cells[:, :_TOPK].astype(jnp.int32)
    cand_flat = cand.reshape(-1)
    idx2 = (jnp.repeat(cand_flat, _P) * _P
            + jnp.tile(jnp.arange(_P, dtype=jnp.int32), b * _TOPK))
    rows2 = _gather_rows(pm, idx2).reshape(b, _TOPK, _P, d)
    cm = _gather_rows(fm, cand_flat)
    bd, bmsel = _proto_argmin(rows2, embedding, cm)
    rows3 = _gather_rows(embeddings, bmsel.reshape(-1))
    rows3 = rows3.reshape(b, _TOPK, _M, d)
    bg = _member_argmin(rows3, embedding, bmsel)
    llpad = jnp.pad(dataset_latlng, ((0, 0), (0, 14)))
    crows = _gather_rows(llpad, bg.reshape(-1)).reshape(b, _TOPK, 16)
    latk = crows[:, :, 0]
    lngk = crows[:, :, 1]
    return _finish(bd, candidate_probs[:, :_TOPK], cand, latk, lngk,
                   initial_preds, jnp.reshape(temperature, (1, 1)))


# speedup vs baseline: 1.0074x; 1.0074x over previous
import functools
import math

import jax
import jax.numpy as jnp
from jax import lax
from jax.experimental import pallas as pl
from jax.experimental.pallas import tpu as pltpu
from jax.experimental.pallas import tpu_sc as plsc

_D = 512
_G = 1000
_P = 8
_M = 16
_TOPK = 5
_ATHR = math.sin(1000.0 / (2.0 * 6371.0)) ** 2

_NC = 2
_NS = 16
_NW = _NC * _NS
_LANES = 16


def _pick_chunk(rpw, d):
    best = 8
    for c in range(8, 129, 8):
        if rpw % c == 0 and c * d * 4 <= 163840:
            best = c
    return best


def _gather_rows(table, idx, *, unroll=10):
    n, = idx.shape
    _, d = table.shape
    rpw = n // _NW
    assert rpw * _NW == n and rpw % 8 == 0
    chunk = _pick_chunk(rpw, d)
    nchunks = rpw // chunk
    mesh = plsc.VectorSubcoreMesh(core_axis_name="c", subcore_axis_name="s")

    @functools.partial(
        pl.kernel,
        mesh=mesh,
        compiler_params=pltpu.CompilerParams(use_tc_tiling_on_sc=(d % 128 == 0)),
        out_type=jax.ShapeDtypeStruct((n, d), table.dtype),
        scratch_types=[
            pltpu.VMEM((rpw,), jnp.int32),
            pltpu.VMEM((chunk, d), table.dtype),
            pltpu.VMEM((chunk, d), table.dtype),
            pltpu.SemaphoreType.DMA,
            pltpu.SemaphoreType.DMA,
            pltpu.SemaphoreType.DMA,
            pltpu.SemaphoreType.DMA,
        ],
    )
    def k(table_hbm, idx_hbm, out_hbm, idx_v, buf0, buf1,
          gsem0, gsem1, ssem0, ssem1):
        wid = lax.axis_index("s") * _NC + lax.axis_index("c")
        base = wid * rpw
        pltpu.sync_copy(idx_hbm.at[pl.ds(base, rpw)], idx_v)
        bufs = (buf0, buf1)
        gsems = (gsem0, gsem1)
        ssems = (ssem0, ssem1)

        def run_group(g, nin):
            gcp = {}
            scp = {}

            def start_gather(j):
                ci = g * unroll + j
                gcp[j] = pltpu.async_copy(
                    table_hbm.at[idx_v.at[pl.ds(ci * chunk, chunk)]],
                    bufs[j % 2],
                    gsems[j % 2],
                )

            start_gather(0)
            for j in range(nin):
                if j + 1 < nin:
                    if j - 1 >= 0:
                        scp[j - 1].wait()
                    start_gather(j + 1)
                gcp[j].wait()
                ci = g * unroll + j
                scp[j] = pltpu.async_copy(
                    bufs[j % 2],
                    out_hbm.at[pl.ds(base + ci * chunk, chunk)],
                    ssems[j % 2],
                )
            if nin - 2 >= 0:
                scp[nin - 2].wait()
            scp[nin - 1].wait()

        full, rem = divmod(nchunks, unroll)
        if full > 1:
            def body(g, c):
                run_group(g, unroll)
                return c
            lax.fori_loop(0, full, body, 0)
        elif full == 1:
            run_group(0, unroll)
        if rem:
            run_group(full, rem)

    return k(table, idx)


def _gather_mean16(table, idx):
    n, = idx.shape
    _, d = table.shape
    nprotos = n // _M
    ppw = nprotos // _NW
    rpw = n // _NW
    cp = 4
    chunk = cp * _M
    spg = 32
    nsg = ppw // spg
    gpsg = spg // (2 * cp)
    nlc = d // _LANES
    mesh = plsc.VectorSubcoreMesh(core_axis_name="c", subcore_axis_name="s")

    @functools.partial(
        pl.kernel,
        mesh=mesh,
        out_type=jax.ShapeDtypeStruct((nprotos, d), jnp.float32),
        scratch_types=[
            pltpu.VMEM((rpw,), jnp.int32),
            pltpu.VMEM((chunk, d), jnp.float32),
            pltpu.VMEM((chunk, d), jnp.float32),
            pltpu.VMEM((spg, d), jnp.float32),
            pltpu.VMEM((spg, d), jnp.float32),
            pltpu.SemaphoreType.DMA,
            pltpu.SemaphoreType.DMA,
            pltpu.SemaphoreType.DMA,
            pltpu.SemaphoreType.DMA,
        ],
    )
    def k(table_hbm, idx_hbm, out_hbm, idx_v, buf0, buf1, mb0, mb1,
          gsem0, gsem1, msem0, msem1):
        wid = lax.axis_index("s") * _NC + lax.axis_index("c")
        base = wid * rpw
        pbase = wid * ppw
        pltpu.sync_copy(idx_hbm.at[pl.ds(base, rpw)], idx_v)
        bufs = (buf0, buf1)
        mbufs = (mb0, mb1)
        gsems = (gsem0, gsem1)
        msems = (msem0, msem1)

        def start_gather(ci, side):
            pltpu.async_copy(
                table_hbm.at[idx_v.at[pl.ds(ci * chunk, chunk)]],
                bufs[side],
                gsems[side],
            )

        def wait_gather(side):
            pltpu.make_async_copy(
                table_hbm.at[idx_v.at[pl.ds(0, chunk)]],
                bufs[side],
                gsems[side],
            ).wait()

        def wait_mstore(ms):
            pltpu.make_async_copy(
                mbufs[ms],
                out_hbm.at[pl.ds(pbase, spg)],
                msems[ms],
            ).wait()

        def reduce_chunk(side, mb, mrow0):
            buf = bufs[side]

            def cbody(c, carry):
                col = pl.ds(c * _LANES, _LANES)
                for j in range(cp):
                    r = j * _M
                    v = [buf[r + m, col] for m in range(_M)]
                    t8 = [v[m] + v[m + 8] for m in range(8)]
                    t4 = [t8[m] + t8[m + 4] for m in range(4)]
                    t2 = [t4[m] + t4[m + 2] for m in range(2)]
                    ssum = t2[0] + t2[1]
                    mb[mrow0 + j, col] = ssum * jnp.float32(1.0 / _M)
                return carry
            lax.fori_loop(0, nlc, cbody, 0)

        start_gather(0, 0)
        start_gather(1, 1)

        def run_sg(sg, ms):
            def gbody(g2, carry):
                ci0 = sg * 2 * gpsg + 2 * g2
                wait_gather(0)
                reduce_chunk(0, mbufs[ms], g2 * 2 * cp)
                @pl.when(ci0 + 2 < 2 * gpsg * nsg)
                def _():
                    start_gather(ci0 + 2, 0)
                wait_gather(1)
                reduce_chunk(1, mbufs[ms], g2 * 2 * cp + cp)
                @pl.when(ci0 + 3 < 2 * gpsg * nsg)
                def _():
                    start_gather(ci0 + 3, 1)
                return carry
            lax.fori_loop(0, gpsg, gbody, 0)
            pltpu.async_copy(
                mbufs[ms],
                out_hbm.at[pl.ds(pbase + sg * spg, spg)],
                msems[ms],
            )

        def sgbody(sg, carry):
            @pl.when(lax.rem(sg, 2) == 0)
            def _():
                @pl.when(sg >= 2)
                def _():
                    wait_mstore(0)
                run_sg(sg, 0)
            @pl.when(lax.rem(sg, 2) == 1)
            def _():
                @pl.when(sg >= 2)
                def _():
                    wait_mstore(1)
                run_sg(sg, 1)
            return carry

        lax.fori_loop(0, nsg, sgbody, 0)
        if nsg >= 2:
            wait_mstore(0)
        wait_mstore(1)

    return k(table, idx)


def _repack_members(pi3):
    def kfn(i_ref, o_ref):
        arr = i_ref[...]
        for p in range(_P):
            o_ref[:, p * _M:(p + 1) * _M] = arr[:, p, :]

    return pl.pallas_call(
        kfn,
        out_shape=jax.ShapeDtypeStruct((_G, _P * _M), jnp.int32),
    )(pi3)


def _proto_argmin(rows2, emb, cm):
    b = emb.shape[0]
    bb = 64

    def kfn(r_ref, e_ref, cm_ref, bd_ref, bm_ref):
        r = r_ref[...]
        e = e_ref[...]
        dd = r - e[:, None, None, :]
        s = jnp.sqrt(jnp.sum(dd * dd, axis=-1) + 1e-12)
        best = s[:, :, 0]
        bi = jnp.zeros(best.shape, jnp.int32)
        for p in range(1, _P):
            c = s[:, :, p]
            lt = c < best
            bi = jnp.where(lt, p, bi)
            best = jnp.where(lt, c, best)
        cmv = cm_ref[...].reshape(bb, _TOPK, _P * _M)
        bm = cmv[:, :, 0:_M]
        for p in range(1, _P):
            sel = (bi == p)[:, :, None]
            bm = jnp.where(sel, cmv[:, :, p * _M:(p + 1) * _M], bm)
        bd_ref[...] = best
        bm_ref[...] = bm

    return pl.pallas_call(
        kfn,
        grid=(b // bb,),
        in_specs=[
            pl.BlockSpec((bb, _TOPK, _P, _D), lambda i: (i, 0, 0, 0)),
            pl.BlockSpec((bb, _D), lambda i: (i, 0)),
            pl.BlockSpec((bb * _TOPK, _P * _M), lambda i: (i, 0)),
        ],
        out_specs=[
            pl.BlockSpec((bb, _TOPK), lambda i: (i, 0)),
            pl.BlockSpec((bb, _TOPK, _M), lambda i: (i, 0, 0)),
        ],
        out_shape=[
            jax.ShapeDtypeStruct((b, _TOPK), jnp.float32),
            jax.ShapeDtypeStruct((b, _TOPK, _M), jnp.int32),
        ],
    )(rows2, emb, cm)


def _member_argmin(rows3, emb, bm):
    b = emb.shape[0]
    bb = 64

    def kfn(r_ref, e_ref, bm_ref, bg_ref):
        r = r_ref[...]
        e = e_ref[...]
        dd = r - e[:, None, None, :]
        s = jnp.sqrt(jnp.sum(dd * dd, axis=-1) + 1e-12)
        best = s[:, :, 0]
        bi = jnp.zeros(best.shape, jnp.int32)
        for m in range(1, _M):
            c = s[:, :, m]
            lt = c < best
            bi = jnp.where(lt, m, bi)
            best = jnp.where(lt, c, best)
        bmv = bm_ref[...]
        bg = bmv[:, :, 0]
        for m in range(1, _M):
            bg = jnp.where(bi == m, bmv[:, :, m], bg)
        bg_ref[...] = bg

    return pl.pallas_call(
        kfn,
        grid=(b // bb,),
        in_specs=[
            pl.BlockSpec((bb, _TOPK, _M, _D), lambda i: (i, 0, 0, 0)),
            pl.BlockSpec((bb, _D), lambda i: (i, 0)),
            pl.BlockSpec((bb, _TOPK, _M), lambda i: (i, 0, 0)),
        ],
        out_specs=pl.BlockSpec((bb, _TOPK), lambda i: (i, 0)),
        out_shape=jax.ShapeDtypeStruct((b, _TOPK), jnp.int32),
    )(rows3, emb, bm)


def _finish(bd, cp5, cand5, latk, lngk, ipreds, temp):
    b = bd.shape[0]

    def kfn(bd_ref, cp_ref, cd_ref, la_ref, lo_ref, ip_ref, t_ref,
            olat_ref, olng_ref, ogc_ref):
        scores = -bd_ref[...]
        t = t_ref[0, 0]
        ex = jnp.exp(scores / t)
        probs = ex / jnp.sum(ex, axis=-1, keepdims=True)
        fp = cp_ref[...] * probs
        best = fp[:, 0]
        bi = jnp.zeros(best.shape, jnp.int32)
        for k2 in range(1, _TOPK):
            c = fp[:, k2]
            gt = c > best
            bi = jnp.where(gt, k2, bi)
            best = jnp.where(gt, c, best)
        la = la_ref[...]
        lo = lo_ref[...]
        rlat = la[:, 0]
        rlng = lo[:, 0]
        for k2 in range(1, _TOPK):
            rlat = jnp.where(bi == k2, la[:, k2], rlat)
            rlng = jnp.where(bi == k2, lo[:, k2], rlng)
        deg = jnp.float32(math.pi / 180.0)
        lat1 = ip_ref[:, 0] * deg
        lng1 = ip_ref[:, 1] * deg
        lat2 = rlat * deg
        lng2 = rlng * deg
        sdlat = jnp.sin((lat2 - lat1) * 0.5)
        sdlng = jnp.sin((lng2 - lng1) * 0.5)
        a = sdlat * sdlat + jnp.cos(lat1) * jnp.cos(lat2) * sdlng * sdlng
        fi = jnp.where(a > jnp.float32(_ATHR), 0, bi)
        flat = la[:, 0]
        flng = lo[:, 0]
        cd = cd_ref[...]
        gc = cd[:, 0]
        for k2 in range(1, _TOPK):
            sel = fi == k2
            flat = jnp.where(sel, la[:, k2], flat)
            flng = jnp.where(sel, lo[:, k2], flng)
            gc = jnp.where(sel, cd[:, k2], gc)
        olat_ref[...] = flat
        olng_ref[...] = flng
        ogc_ref[...] = gc

    return pl.pallas_call(
        kfn,
        out_shape=[
            jax.ShapeDtypeStruct((b,), jnp.float32),
            jax.ShapeDtypeStruct((b,), jnp.float32),
            jax.ShapeDtypeStruct((b,), cand5.dtype),
        ],
    )(bd, cp5, cand5, latk, lngk, ipreds, temp)


def kernel(embedding, initial_preds, candidate_cells, candidate_probs,
           embeddings, proto_indices, dataset_latlng, temperature):
    b, d = embedding.shape
    fm = _repack_members(proto_indices.astype(jnp.int32))
    idx1 = jnp.pad(fm.reshape(-1), (0, 256 * _NW * _M - _G * _P * _M))
    pm = _gather_mean16(embeddings, idx1)
    cand = candidate_cells[:, :_TOPK].astype(jnp.int32)
    cand_flat = cand.reshape(-1)
    idx2 = (jnp.repeat(cand_flat, _P) * _P
            + jnp.tile(jnp.arange(_P, dtype=jnp.int32), b * _TOPK))
    rows2 = _gather_rows(pm, idx2).reshape(b, _TOPK, _P, d)
    cm = _gather_rows(fm, cand_flat)
    bd, bmsel = _proto_argmin(rows2, embedding, cm)
    rows3 = _gather_rows(embeddings, bmsel.reshape(-1))
    rows3 = rows3.reshape(b, _TOPK, _M, d)
    bg = _member_argmin(rows3, embedding, bmsel)
    llpad = jnp.pad(dataset_latlng, ((0, 0), (0, 14)))
    crows = _gather_rows(llpad, bg.reshape(-1)).reshape(b, _TOPK, 16)
    latk = crows[:, :, 0]
    lngk = crows[:, :, 1]
    return _finish(bd, candidate_probs[:, :_TOPK], cand, latk, lngk,
                   initial_preds, jnp.reshape(temperature, (1, 1)))


# speedup vs baseline: 1.2539x; 1.2447x over previous
import functools
import math

import jax
import jax.numpy as jnp
from jax import lax
from jax.experimental import pallas as pl
from jax.experimental.pallas import tpu as pltpu
from jax.experimental.pallas import tpu_sc as plsc

_D = 512
_G = 1000
_P = 8
_M = 16
_TOPK = 5
_ATHR = math.sin(1000.0 / (2.0 * 6371.0)) ** 2

_NC = 2
_NS = 16
_NW = _NC * _NS
_LANES = 16


def _pick_chunk(rpw, d):
    best = 8
    for c in range(8, 129, 8):
        if rpw % c == 0 and c * d * 4 <= 163840:
            best = c
    return best


def _gather_rows(table, idx, *, unroll=10):
    n, = idx.shape
    _, d = table.shape
    rpw = n // _NW
    assert rpw * _NW == n and rpw % 8 == 0
    chunk = _pick_chunk(rpw, d)
    nchunks = rpw // chunk
    mesh = plsc.VectorSubcoreMesh(core_axis_name="c", subcore_axis_name="s")

    @functools.partial(
        pl.kernel,
        mesh=mesh,
        compiler_params=pltpu.CompilerParams(use_tc_tiling_on_sc=(d % 128 == 0)),
        out_type=jax.ShapeDtypeStruct((n, d), table.dtype),
        scratch_types=[
            pltpu.VMEM((rpw,), jnp.int32),
            pltpu.VMEM((chunk, d), table.dtype),
            pltpu.VMEM((chunk, d), table.dtype),
            pltpu.SemaphoreType.DMA,
            pltpu.SemaphoreType.DMA,
            pltpu.SemaphoreType.DMA,
            pltpu.SemaphoreType.DMA,
        ],
    )
    def k(table_hbm, idx_hbm, out_hbm, idx_v, buf0, buf1,
          gsem0, gsem1, ssem0, ssem1):
        wid = lax.axis_index("s") * _NC + lax.axis_index("c")
        base = wid * rpw
        pltpu.sync_copy(idx_hbm.at[pl.ds(base, rpw)], idx_v)
        bufs = (buf0, buf1)
        gsems = (gsem0, gsem1)
        ssems = (ssem0, ssem1)

        def run_group(g, nin):
            gcp = {}
            scp = {}

            def start_gather(j):
                ci = g * unroll + j
                gcp[j] = pltpu.async_copy(
                    table_hbm.at[idx_v.at[pl.ds(ci * chunk, chunk)]],
                    bufs[j % 2],
                    gsems[j % 2],
                )

            start_gather(0)
            for j in range(nin):
                if j + 1 < nin:
                    if j - 1 >= 0:
                        scp[j - 1].wait()
                    start_gather(j + 1)
                gcp[j].wait()
                ci = g * unroll + j
                scp[j] = pltpu.async_copy(
                    bufs[j % 2],
                    out_hbm.at[pl.ds(base + ci * chunk, chunk)],
                    ssems[j % 2],
                )
            if nin - 2 >= 0:
                scp[nin - 2].wait()
            scp[nin - 1].wait()

        full, rem = divmod(nchunks, unroll)
        if full > 1:
            def body(g, c):
                run_group(g, unroll)
                return c
            lax.fori_loop(0, full, body, 0)
        elif full == 1:
            run_group(0, unroll)
        if rem:
            run_group(full, rem)

    return k(table, idx)


def _gather_mean16(table, idx):
    n, = idx.shape
    _, d = table.shape
    nprotos = n // _M
    ppw = nprotos // _NW
    rpw = n // _NW
    cp = 4
    chunk = cp * _M
    spg = 32
    nsg = ppw // spg
    gpsg = spg // (2 * cp)
    nlc = d // _LANES
    mesh = plsc.VectorSubcoreMesh(core_axis_name="c", subcore_axis_name="s")

    @functools.partial(
        pl.kernel,
        mesh=mesh,
        out_type=jax.ShapeDtypeStruct((nprotos, d), jnp.float32),
        scratch_types=[
            pltpu.VMEM((rpw,), jnp.int32),
            pltpu.VMEM((chunk, d), jnp.float32),
            pltpu.VMEM((chunk, d), jnp.float32),
            pltpu.VMEM((spg, d), jnp.float32),
            pltpu.VMEM((spg, d), jnp.float32),
            pltpu.SemaphoreType.DMA,
            pltpu.SemaphoreType.DMA,
            pltpu.SemaphoreType.DMA,
            pltpu.SemaphoreType.DMA,
        ],
    )
    def k(table_hbm, idx_hbm, out_hbm, idx_v, buf0, buf1, mb0, mb1,
          gsem0, gsem1, msem0, msem1):
        wid = lax.axis_index("s") * _NC + lax.axis_index("c")
        base = wid * rpw
        pbase = wid * ppw
        pltpu.sync_copy(idx_hbm.at[pl.ds(base, rpw)], idx_v)
        bufs = (buf0, buf1)
        mbufs = (mb0, mb1)
        gsems = (gsem0, gsem1)
        msems = (msem0, msem1)

        def start_gather(ci, side):
            pltpu.async_copy(
                table_hbm.at[idx_v.at[pl.ds(ci * chunk, chunk)]],
                bufs[side],
                gsems[side],
            )

        def wait_gather(side):
            pltpu.make_async_copy(
                table_hbm.at[idx_v.at[pl.ds(0, chunk)]],
                bufs[side],
                gsems[side],
            ).wait()

        def wait_mstore(ms):
            pltpu.make_async_copy(
                mbufs[ms],
                out_hbm.at[pl.ds(pbase, spg)],
                msems[ms],
            ).wait()

        def reduce_chunk(side, mb, mrow0):
            buf = bufs[side]

            def cbody(c, carry):
                col = pl.ds(c * _LANES, _LANES)
                for j in range(cp):
                    r = j * _M
                    v = [buf[r + m, col] for m in range(_M)]
                    t8 = [v[m] + v[m + 8] for m in range(8)]
                    t4 = [t8[m] + t8[m + 4] for m in range(4)]
                    t2 = [t4[m] + t4[m + 2] for m in range(2)]
                    ssum = t2[0] + t2[1]
                    mb[mrow0 + j, col] = ssum * jnp.float32(1.0 / _M)
                return carry
            lax.fori_loop(0, nlc, cbody, 0)

        start_gather(0, 0)
        start_gather(1, 1)

        def run_sg(sg, ms):
            def gbody(g2, carry):
                ci0 = sg * 2 * gpsg + 2 * g2
                wait_gather(0)
                reduce_chunk(0, mbufs[ms], g2 * 2 * cp)
                @pl.when(ci0 + 2 < 2 * gpsg * nsg)
                def _():
                    start_gather(ci0 + 2, 0)
                wait_gather(1)
                reduce_chunk(1, mbufs[ms], g2 * 2 * cp + cp)
                @pl.when(ci0 + 3 < 2 * gpsg * nsg)
                def _():
                    start_gather(ci0 + 3, 1)
                return carry
            lax.fori_loop(0, gpsg, gbody, 0)
            pltpu.async_copy(
                mbufs[ms],
                out_hbm.at[pl.ds(pbase + sg * spg, spg)],
                msems[ms],
            )

        def sgbody(sg, carry):
            @pl.when(lax.rem(sg, 2) == 0)
            def _():
                @pl.when(sg >= 2)
                def _():
                    wait_mstore(0)
                run_sg(sg, 0)
            @pl.when(lax.rem(sg, 2) == 1)
            def _():
                @pl.when(sg >= 2)
                def _():
                    wait_mstore(1)
                run_sg(sg, 1)
            return carry

        lax.fori_loop(0, nsg, sgbody, 0)
        if nsg >= 2:
            wait_mstore(0)
        wait_mstore(1)

    return k(table, idx)


def _repack_members(pi3):
    def kfn(i_ref, o_ref):
        arr = i_ref[...]
        for p in range(_P):
            o_ref[:, p * _M:(p + 1) * _M] = arr[:, p, :]

    return pl.pallas_call(
        kfn,
        out_shape=jax.ShapeDtypeStruct((_G, _P * _M), jnp.int32),
    )(pi3)


def _proto_argmin(rows2, emb, cm):
    b = emb.shape[0]
    bb = 64

    def kfn(r_ref, e_ref, cm_ref, bd_ref, bm_ref):
        r = r_ref[...]
        e = e_ref[...]
        dd = r - e[:, None, None, :]
        s = jnp.sqrt(jnp.sum(dd * dd, axis=-1) + 1e-12)
        best = jnp.min(s, axis=-1)
        i8 = lax.broadcasted_iota(jnp.int32, s.shape, 2)
        bi = jnp.min(jnp.where(s == best[..., None], i8, _P), axis=-1)
        cmv4 = cm_ref[...].reshape(bb, _TOPK, _P, _M)
        sel = i8[..., None] == bi[..., None, None]
        bm = jnp.sum(jnp.where(sel, cmv4, 0), axis=2)
        bd_ref[...] = best
        bm_ref[...] = bm

    return pl.pallas_call(
        kfn,
        grid=(b // bb,),
        in_specs=[
            pl.BlockSpec((bb, _TOPK, _P, _D), lambda i: (i, 0, 0, 0)),
            pl.BlockSpec((bb, _D), lambda i: (i, 0)),
            pl.BlockSpec((bb * _TOPK, _P * _M), lambda i: (i, 0)),
        ],
        out_specs=[
            pl.BlockSpec((bb, _TOPK), lambda i: (i, 0)),
            pl.BlockSpec((bb, _TOPK, _M), lambda i: (i, 0, 0)),
        ],
        out_shape=[
            jax.ShapeDtypeStruct((b, _TOPK), jnp.float32),
            jax.ShapeDtypeStruct((b, _TOPK, _M), jnp.int32),
        ],
    )(rows2, emb, cm)


def _member_argmin(rows3, emb, bm):
    b = emb.shape[0]
    bb = 64

    def kfn(r_ref, e_ref, bm_ref, bg_ref):
        r = r_ref[...]
        e = e_ref[...]
        dd = r - e[:, None, None, :]
        s = jnp.sqrt(jnp.sum(dd * dd, axis=-1) + 1e-12)
        best = jnp.min(s, axis=-1)
        i16 = lax.broadcasted_iota(jnp.int32, s.shape, 2)
        bi = jnp.min(jnp.where(s == best[..., None], i16, _M), axis=-1)
        bmv = bm_ref[...]
        bg = jnp.sum(jnp.where(i16 == bi[..., None], bmv, 0), axis=-1)
        bg_ref[...] = bg

    return pl.pallas_call(
        kfn,
        grid=(b // bb,),
        in_specs=[
            pl.BlockSpec((bb, _TOPK, _M, _D), lambda i: (i, 0, 0, 0)),
            pl.BlockSpec((bb, _D), lambda i: (i, 0)),
            pl.BlockSpec((bb, _TOPK, _M), lambda i: (i, 0, 0)),
        ],
        out_specs=pl.BlockSpec((bb, _TOPK), lambda i: (i, 0)),
        out_shape=jax.ShapeDtypeStruct((b, _TOPK), jnp.int32),
    )(rows3, emb, bm)


def _finish(bd, cp5, cand5, latk, lngk, ipreds, temp):
    b = bd.shape[0]

    def kfn(bd_ref, cp_ref, cd_ref, la_ref, lo_ref, ip_ref, t_ref,
            olat_ref, olng_ref, ogc_ref):
        scores = -bd_ref[...]
        t = t_ref[0, 0]
        ex = jnp.exp(scores / t)
        probs = ex / jnp.sum(ex, axis=-1, keepdims=True)
        fp = cp_ref[...] * probs
        i5 = lax.broadcasted_iota(jnp.int32, fp.shape, 1)
        mx = jnp.max(fp, axis=-1)
        bi = jnp.min(jnp.where(fp == mx[:, None], i5, _TOPK), axis=-1)
        la = la_ref[...]
        lo = lo_ref[...]
        selb = i5 == bi[:, None]
        rlat = jnp.sum(jnp.where(selb, la, jnp.float32(0)), axis=-1)
        rlng = jnp.sum(jnp.where(selb, lo, jnp.float32(0)), axis=-1)
        deg = jnp.float32(math.pi / 180.0)
        lat1 = ip_ref[:, 0] * deg
        lng1 = ip_ref[:, 1] * deg
        lat2 = rlat * deg
        lng2 = rlng * deg
        sdlat = jnp.sin((lat2 - lat1) * 0.5)
        sdlng = jnp.sin((lng2 - lng1) * 0.5)
        a = sdlat * sdlat + jnp.cos(lat1) * jnp.cos(lat2) * sdlng * sdlng
        fi = jnp.where(a > jnp.float32(_ATHR), 0, bi)
        cd = cd_ref[...]

        self_ = i5 == fi[:, None]
        flat = jnp.sum(jnp.where(self_, la, jnp.float32(0)), axis=-1)
        flng = jnp.sum(jnp.where(self_, lo, jnp.float32(0)), axis=-1)
        gc = jnp.sum(jnp.where(self_, cd, 0), axis=-1)
        olat_ref[...] = flat
        olng_ref[...] = flng
        ogc_ref[...] = gc

    return pl.pallas_call(
        kfn,
        out_shape=[
            jax.ShapeDtypeStruct((b,), jnp.float32),
            jax.ShapeDtypeStruct((b,), jnp.float32),
            jax.ShapeDtypeStruct((b,), cand5.dtype),
        ],
    )(bd, cp5, cand5, latk, lngk, ipreds, temp)


def kernel(embedding, initial_preds, candidate_cells, candidate_probs,
           embeddings, proto_indices, dataset_latlng, temperature):
    b, d = embedding.shape
    fm = _repack_members(proto_indices.astype(jnp.int32))
    idx1 = jnp.pad(fm.reshape(-1), (0, 256 * _NW * _M - _G * _P * _M))
    pm = _gather_mean16(embeddings, idx1)
    cand = candidate_cells[:, :_TOPK].astype(jnp.int32)
    cand_flat = cand.reshape(-1)
    idx2 = (jnp.repeat(cand_flat, _P) * _P
            + jnp.tile(jnp.arange(_P, dtype=jnp.int32), b * _TOPK))
    rows2 = _gather_rows(pm, idx2).reshape(b, _TOPK, _P, d)
    cm = _gather_rows(fm, cand_flat)
    bd, bmsel = _proto_argmin(rows2, embedding, cm)
    rows3 = _gather_rows(embeddings, bmsel.reshape(-1))
    rows3 = rows3.reshape(b, _TOPK, _M, d)
    bg = _member_argmin(rows3, embedding, bmsel)
    llpad = jnp.pad(dataset_latlng, ((0, 0), (0, 14)))
    crows = _gather_rows(llpad, bg.reshape(-1)).reshape(b, _TOPK, 16)
    latk = crows[:, :, 0]
    lngk = crows[:, :, 1]
    return _finish(bd, candidate_probs[:, :_TOPK], cand, latk, lngk,
                   initial_preds, jnp.reshape(temperature, (1, 1)))


# speedup vs baseline: 1.3311x; 1.0616x over previous
import functools
import math

import jax
import jax.numpy as jnp
from jax import lax
from jax.experimental import pallas as pl
from jax.experimental.pallas import tpu as pltpu
from jax.experimental.pallas import tpu_sc as plsc

_D = 512
_G = 1000
_P = 8
_M = 16
_TOPK = 5
_ATHR = math.sin(1000.0 / (2.0 * 6371.0)) ** 2

_NC = 2
_NS = 16
_NW = _NC * _NS
_LANES = 16


def _pick_chunk(rpw, d):
    best = 8
    for c in range(8, 129, 8):
        if rpw % c == 0 and c * d * 4 <= 163840:
            best = c
    return best


def _gather_rows(table, idx, *, unroll=10):
    n, = idx.shape
    _, d = table.shape
    rpw = n // _NW
    assert rpw * _NW == n and rpw % 8 == 0
    chunk = _pick_chunk(rpw, d)
    nchunks = rpw // chunk
    mesh = plsc.VectorSubcoreMesh(core_axis_name="c", subcore_axis_name="s")

    @functools.partial(
        pl.kernel,
        mesh=mesh,
        compiler_params=pltpu.CompilerParams(use_tc_tiling_on_sc=(d % 128 == 0)),
        out_type=jax.ShapeDtypeStruct((n, d), table.dtype),
        scratch_types=[
            pltpu.VMEM((rpw,), jnp.int32),
            pltpu.VMEM((chunk, d), table.dtype),
            pltpu.VMEM((chunk, d), table.dtype),
            pltpu.SemaphoreType.DMA,
            pltpu.SemaphoreType.DMA,
            pltpu.SemaphoreType.DMA,
            pltpu.SemaphoreType.DMA,
        ],
    )
    def k(table_hbm, idx_hbm, out_hbm, idx_v, buf0, buf1,
          gsem0, gsem1, ssem0, ssem1):
        wid = lax.axis_index("s") * _NC + lax.axis_index("c")
        base = wid * rpw
        pltpu.sync_copy(idx_hbm.at[pl.ds(base, rpw)], idx_v)
        bufs = (buf0, buf1)
        gsems = (gsem0, gsem1)
        ssems = (ssem0, ssem1)

        def run_group(g, nin):
            gcp = {}
            scp = {}

            def start_gather(j):
                ci = g * unroll + j
                gcp[j] = pltpu.async_copy(
                    table_hbm.at[idx_v.at[pl.ds(ci * chunk, chunk)]],
                    bufs[j % 2],
                    gsems[j % 2],
                )

            start_gather(0)
            for j in range(nin):
                if j + 1 < nin:
                    if j - 1 >= 0:
                        scp[j - 1].wait()
                    start_gather(j + 1)
                gcp[j].wait()
                ci = g * unroll + j
                scp[j] = pltpu.async_copy(
                    bufs[j % 2],
                    out_hbm.at[pl.ds(base + ci * chunk, chunk)],
                    ssems[j % 2],
                )
            if nin - 2 >= 0:
                scp[nin - 2].wait()
            scp[nin - 1].wait()

        full, rem = divmod(nchunks, unroll)
        if full > 1:
            def body(g, c):
                run_group(g, unroll)
                return c
            lax.fori_loop(0, full, body, 0)
        elif full == 1:
            run_group(0, unroll)
        if rem:
            run_group(full, rem)

    return k(table, idx)


_K0 = 336
_K1 = 176


def _gather_mean16(table, idx):
    _, d = table.shape
    nprotos = (_K0 + _K1) * _NS
    cp = 4
    chunk = cp * _M
    spg = 16
    gpsg = spg // (2 * cp)
    nlc = d // _LANES
    ppmax = max(_K0, _K1)
    mesh = plsc.VectorSubcoreMesh(core_axis_name="c", subcore_axis_name="s")

    @functools.partial(
        pl.kernel,
        mesh=mesh,
        out_type=jax.ShapeDtypeStruct((nprotos, d), jnp.float32),
        scratch_types=[
            pltpu.VMEM((ppmax * _M,), jnp.int32),
            pltpu.VMEM((chunk, d), jnp.float32),
            pltpu.VMEM((chunk, d), jnp.float32),
            pltpu.VMEM((spg, d), jnp.float32),
            pltpu.VMEM((spg, d), jnp.float32),
            pltpu.SemaphoreType.DMA,
            pltpu.SemaphoreType.DMA,
            pltpu.SemaphoreType.DMA,
            pltpu.SemaphoreType.DMA,
        ],
    )
    def k(table_hbm, idx_hbm, out_hbm, idx_v, buf0, buf1, mb0, mb1,
          gsem0, gsem1, msem0, msem1):
        sx = lax.axis_index("s")
        cx = lax.axis_index("c")
        ppw = jnp.where(cx == 0, _K0, _K1)
        pbase = sx * (_K0 + _K1) + cx * _K0
        nsg = ppw // spg
        ntot = ppw // cp
        pltpu.sync_copy(idx_hbm.at[pl.ds(pbase * _M, ppmax * _M)], idx_v)
        bufs = (buf0, buf1)
        mbufs = (mb0, mb1)
        gsems = (gsem0, gsem1)
        msems = (msem0, msem1)

        def start_gather(ci, side):
            pltpu.async_copy(
                table_hbm.at[idx_v.at[pl.ds(ci * chunk, chunk)]],
                bufs[side],
                gsems[side],
            )

        def wait_gather(side):
            pltpu.make_async_copy(
                table_hbm.at[idx_v.at[pl.ds(0, chunk)]],
                bufs[side],
                gsems[side],
            ).wait()

        def wait_mstore(ms):
            pltpu.make_async_copy(
                mbufs[ms],
                out_hbm.at[pl.ds(pbase, spg)],
                msems[ms],
            ).wait()

        def reduce_chunk(side, mb, mrow0):
            buf = bufs[side]

            def cbody(c, carry):
                col = pl.ds(c * _LANES, _LANES)
                for j in range(cp):
                    r = j * _M
                    v = [buf[r + m, col] for m in range(_M)]
                    t8 = [v[m] + v[m + 8] for m in range(8)]
                    t4 = [t8[m] + t8[m + 4] for m in range(4)]
                    t2 = [t4[m] + t4[m + 2] for m in range(2)]
                    ssum = t2[0] + t2[1]
                    mb[mrow0 + j, col] = ssum * jnp.float32(1.0 / _M)
                return carry
            lax.fori_loop(0, nlc, cbody, 0)

        start_gather(0, 0)
        start_gather(1, 1)

        def run_sg(sg, ms):
            def gbody(g2, carry):
                ci0 = sg * 2 * gpsg + 2 * g2
                wait_gather(0)
                reduce_chunk(0, mbufs[ms], g2 * 2 * cp)
                @pl.when(ci0 + 2 < ntot)
                def _():
                    start_gather(ci0 + 2, 0)
                wait_gather(1)
                reduce_chunk(1, mbufs[ms], g2 * 2 * cp + cp)
                @pl.when(ci0 + 3 < ntot)
                def _():
                    start_gather(ci0 + 3, 1)
                return carry
            lax.fori_loop(0, gpsg, gbody, 0)
            pltpu.async_copy(
                mbufs[ms],
                out_hbm.at[pl.ds(pbase + sg * spg, spg)],
                msems[ms],
            )

        def sgbody(sg, carry):
            @pl.when(lax.rem(sg, 2) == 0)
            def _():
                @pl.when(sg >= 2)
                def _():
                    wait_mstore(0)
                run_sg(sg, 0)
            @pl.when(lax.rem(sg, 2) == 1)
            def _():
                @pl.when(sg >= 2)
                def _():
                    wait_mstore(1)
                run_sg(sg, 1)
            return carry

        lax.fori_loop(0, nsg, sgbody, 0)
        wait_mstore(0)
        wait_mstore(1)

    return k(table, idx)


def _repack_members(pi3):
    def kfn(i_ref, o_ref):
        arr = i_ref[...]
        for p in range(_P):
            o_ref[:, p * _M:(p + 1) * _M] = arr[:, p, :]

    return pl.pallas_call(
        kfn,
        out_shape=jax.ShapeDtypeStruct((_G, _P * _M), jnp.int32),
    )(pi3)


def _proto_argmin(rows2, emb, cm):
    b = emb.shape[0]
    bb = 64

    def kfn(r_ref, e_ref, cm_ref, bd_ref, bm_ref):
        r = r_ref[...]
        e = e_ref[...]
        dd = r - e[:, None, None, :]
        s = jnp.sqrt(jnp.sum(dd * dd, axis=-1) + 1e-12)
        best = jnp.min(s, axis=-1)
        i8 = lax.broadcasted_iota(jnp.int32, s.shape, 2)
        bi = jnp.min(jnp.where(s == best[..., None], i8, _P), axis=-1)
        cmv4 = cm_ref[...].reshape(bb, _TOPK, _P, _M)
        sel = i8[..., None] == bi[..., None, None]
        bm = jnp.sum(jnp.where(sel, cmv4, 0), axis=2)
        bd_ref[...] = best
        bm_ref[...] = bm

    return pl.pallas_call(
        kfn,
        grid=(b // bb,),
        in_specs=[
            pl.BlockSpec((bb, _TOPK, _P, _D), lambda i: (i, 0, 0, 0)),
            pl.BlockSpec((bb, _D), lambda i: (i, 0)),
            pl.BlockSpec((bb * _TOPK, _P * _M), lambda i: (i, 0)),
        ],
        out_specs=[
            pl.BlockSpec((bb, _TOPK), lambda i: (i, 0)),
            pl.BlockSpec((bb, _TOPK, _M), lambda i: (i, 0, 0)),
        ],
        out_shape=[
            jax.ShapeDtypeStruct((b, _TOPK), jnp.float32),
            jax.ShapeDtypeStruct((b, _TOPK, _M), jnp.int32),
        ],
    )(rows2, emb, cm)


def _member_argmin(rows3, emb, bm):
    b = emb.shape[0]
    bb = 64

    def kfn(r_ref, e_ref, bm_ref, bg_ref):
        r = r_ref[...]
        e = e_ref[...]
        dd = r - e[:, None, None, :]
        s = jnp.sqrt(jnp.sum(dd * dd, axis=-1) + 1e-12)
        best = jnp.min(s, axis=-1)
        i16 = lax.broadcasted_iota(jnp.int32, s.shape, 2)
        bi = jnp.min(jnp.where(s == best[..., None], i16, _M), axis=-1)
        bmv = bm_ref[...]
        bg = jnp.sum(jnp.where(i16 == bi[..., None], bmv, 0), axis=-1)
        bg_ref[...] = bg

    return pl.pallas_call(
        kfn,
        grid=(b // bb,),
        in_specs=[
            pl.BlockSpec((bb, _TOPK, _M, _D), lambda i: (i, 0, 0, 0)),
            pl.BlockSpec((bb, _D), lambda i: (i, 0)),
            pl.BlockSpec((bb, _TOPK, _M), lambda i: (i, 0, 0)),
        ],
        out_specs=pl.BlockSpec((bb, _TOPK), lambda i: (i, 0)),
        out_shape=jax.ShapeDtypeStruct((b, _TOPK), jnp.int32),
    )(rows3, emb, bm)


def _finish(bd, cp5, cand5, latk, lngk, ipreds, temp):
    b = bd.shape[0]

    def kfn(bd_ref, cp_ref, cd_ref, la_ref, lo_ref, ip_ref, t_ref,
            olat_ref, olng_ref, ogc_ref):
        scores = -bd_ref[...]
        t = t_ref[0, 0]
        ex = jnp.exp(scores / t)
        probs = ex / jnp.sum(ex, axis=-1, keepdims=True)
        fp = cp_ref[...] * probs
        i5 = lax.broadcasted_iota(jnp.int32, fp.shape, 1)
        mx = jnp.max(fp, axis=-1)
        bi = jnp.min(jnp.where(fp == mx[:, None], i5, _TOPK), axis=-1)
        la = la_ref[...]
        lo = lo_ref[...]
        selb = i5 == bi[:, None]
        rlat = jnp.sum(jnp.where(selb, la, jnp.float32(0)), axis=-1)
        rlng = jnp.sum(jnp.where(selb, lo, jnp.float32(0)), axis=-1)
        deg = jnp.float32(math.pi / 180.0)
        lat1 = ip_ref[:, 0] * deg
        lng1 = ip_ref[:, 1] * deg
        lat2 = rlat * deg
        lng2 = rlng * deg
        sdlat = jnp.sin((lat2 - lat1) * 0.5)
        sdlng = jnp.sin((lng2 - lng1) * 0.5)
        a = sdlat * sdlat + jnp.cos(lat1) * jnp.cos(lat2) * sdlng * sdlng
        fi = jnp.where(a > jnp.float32(_ATHR), 0, bi)
        cd = cd_ref[...]

        self_ = i5 == fi[:, None]
        flat = jnp.sum(jnp.where(self_, la, jnp.float32(0)), axis=-1)
        flng = jnp.sum(jnp.where(self_, lo, jnp.float32(0)), axis=-1)
        gc = jnp.sum(jnp.where(self_, cd, 0), axis=-1)
        olat_ref[...] = flat
        olng_ref[...] = flng
        ogc_ref[...] = gc

    return pl.pallas_call(
        kfn,
        out_shape=[
            jax.ShapeDtypeStruct((b,), jnp.float32),
            jax.ShapeDtypeStruct((b,), jnp.float32),
            jax.ShapeDtypeStruct((b,), cand5.dtype),
        ],
    )(bd, cp5, cand5, latk, lngk, ipreds, temp)


def kernel(embedding, initial_preds, candidate_cells, candidate_probs,
           embeddings, proto_indices, dataset_latlng, temperature):
    b, d = embedding.shape
    fm = _repack_members(proto_indices.astype(jnp.int32))
    npad = ((_K0 + _K1) * _NS - _G * _P + max(_K0, _K1)) * _M
    idx1 = jnp.pad(fm.reshape(-1), (0, npad))
    pm = _gather_mean16(embeddings, idx1)
    cand = candidate_cells[:, :_TOPK].astype(jnp.int32)
    cand_flat = cand.reshape(-1)
    idx2 = (jnp.repeat(cand_flat, _P) * _P
            + jnp.tile(jnp.arange(_P, dtype=jnp.int32), b * _TOPK))
    rows2 = _gather_rows(pm, idx2).reshape(b, _TOPK, _P, d)
    cm = _gather_rows(fm, cand_flat)
    bd, bmsel = _proto_argmin(rows2, embedding, cm)
    rows3 = _gather_rows(embeddings, bmsel.reshape(-1))
    rows3 = rows3.reshape(b, _TOPK, _M, d)
    bg = _member_argmin(rows3, embedding, bmsel)
    llpad = jnp.pad(dataset_latlng, ((0, 0), (0, 14)))
    crows = _gather_rows(llpad, bg.reshape(-1)).reshape(b, _TOPK, 16)
    latk = crows[:, :, 0]
    lngk = crows[:, :, 1]
    return _finish(bd, candidate_probs[:, :_TOPK], cand, latk, lngk,
                   initial_preds, jnp.reshape(temperature, (1, 1)))


# speedup vs baseline: 1.3424x; 1.0085x over previous
import functools
import math

import jax
import jax.numpy as jnp
from jax import lax
from jax.experimental import pallas as pl
from jax.experimental.pallas import tpu as pltpu
from jax.experimental.pallas import tpu_sc as plsc

_D = 512
_G = 1000
_P = 8
_M = 16
_TOPK = 5
_ATHR = math.sin(1000.0 / (2.0 * 6371.0)) ** 2

_NC = 2
_NS = 16
_NW = _NC * _NS
_LANES = 16


def _pick_chunk(rpw, d):
    best = 8
    for c in range(8, 129, 8):
        if rpw % c == 0 and c * d * 4 <= 163840:
            best = c
    return best


def _gather_rows(table, idx, *, unroll=10):
    n, = idx.shape
    _, d = table.shape
    rpw = n // _NW
    assert rpw * _NW == n and rpw % 8 == 0
    chunk = _pick_chunk(rpw, d)
    nchunks = rpw // chunk
    mesh = plsc.VectorSubcoreMesh(core_axis_name="c", subcore_axis_name="s")

    @functools.partial(
        pl.kernel,
        mesh=mesh,
        compiler_params=pltpu.CompilerParams(use_tc_tiling_on_sc=(d % 128 == 0)),
        out_type=jax.ShapeDtypeStruct((n, d), table.dtype),
        scratch_types=[
            pltpu.VMEM((rpw,), jnp.int32),
            pltpu.VMEM((chunk, d), table.dtype),
            pltpu.VMEM((chunk, d), table.dtype),
            pltpu.SemaphoreType.DMA,
            pltpu.SemaphoreType.DMA,
            pltpu.SemaphoreType.DMA,
            pltpu.SemaphoreType.DMA,
        ],
    )
    def k(table_hbm, idx_hbm, out_hbm, idx_v, buf0, buf1,
          gsem0, gsem1, ssem0, ssem1):
        wid = lax.axis_index("s") * _NC + lax.axis_index("c")
        base = wid * rpw
        pltpu.sync_copy(idx_hbm.at[pl.ds(base, rpw)], idx_v)
        bufs = (buf0, buf1)
        gsems = (gsem0, gsem1)
        ssems = (ssem0, ssem1)

        def run_group(g, nin):
            gcp = {}
            scp = {}

            def start_gather(j):
                ci = g * unroll + j
                gcp[j] = pltpu.async_copy(
                    table_hbm.at[idx_v.at[pl.ds(ci * chunk, chunk)]],
                    bufs[j % 2],
                    gsems[j % 2],
                )

            start_gather(0)
            for j in range(nin):
                if j + 1 < nin:
                    if j - 1 >= 0:
                        scp[j - 1].wait()
                    start_gather(j + 1)
                gcp[j].wait()
                ci = g * unroll + j
                scp[j] = pltpu.async_copy(
                    bufs[j % 2],
                    out_hbm.at[pl.ds(base + ci * chunk, chunk)],
                    ssems[j % 2],
                )
            if nin - 2 >= 0:
                scp[nin - 2].wait()
            scp[nin - 1].wait()

        full, rem = divmod(nchunks, unroll)
        if full > 1:
            def body(g, c):
                run_group(g, unroll)
                return c
            lax.fori_loop(0, full, body, 0)
        elif full == 1:
            run_group(0, unroll)
        if rem:
            run_group(full, rem)

    return k(table, idx)


_K0 = 336
_K1 = 176


def _gather_mean16(table, idx):
    _, d = table.shape
    nprotos = (_K0 + _K1) * _NS
    cp = 4
    chunk = cp * _M
    spg = 16
    gpsg = spg // (2 * cp)
    nlc = d // _LANES
    ppmax = max(_K0, _K1)
    mesh = plsc.VectorSubcoreMesh(core_axis_name="c", subcore_axis_name="s")

    @functools.partial(
        pl.kernel,
        mesh=mesh,
        out_type=jax.ShapeDtypeStruct((nprotos, d), jnp.float32),
        scratch_types=[
            pltpu.VMEM((ppmax * _M,), jnp.int32),
            pltpu.VMEM((chunk, d), jnp.float32),
            pltpu.VMEM((chunk, d), jnp.float32),
            pltpu.VMEM((spg, d), jnp.float32),
            pltpu.VMEM((spg, d), jnp.float32),
            pltpu.SemaphoreType.DMA,
            pltpu.SemaphoreType.DMA,
            pltpu.SemaphoreType.DMA,
            pltpu.SemaphoreType.DMA,
        ],
    )
    def k(table_hbm, idx_hbm, out_hbm, idx_v, buf0, buf1, mb0, mb1,
          gsem0, gsem1, msem0, msem1):
        sx = lax.axis_index("s")
        cx = lax.axis_index("c")
        ppw = jnp.where(cx == 0, _K0, _K1)
        pbase = sx * (_K0 + _K1) + cx * _K0
        nsg = ppw // spg
        ntot = ppw // cp
        pltpu.sync_copy(idx_hbm.at[pl.ds(pbase * _M, ppmax * _M)], idx_v)
        bufs = (buf0, buf1)
        mbufs = (mb0, mb1)
        gsems = (gsem0, gsem1)
        msems = (msem0, msem1)

        def start_gather(ci, side):
            pltpu.async_copy(
                table_hbm.at[idx_v.at[pl.ds(ci * chunk, chunk)]],
                bufs[side],
                gsems[side],
            )

        def wait_gather(side):
            pltpu.make_async_copy(
                table_hbm.at[idx_v.at[pl.ds(0, chunk)]],
                bufs[side],
                gsems[side],
            ).wait()

        def wait_mstore(ms):
            pltpu.make_async_copy(
                mbufs[ms],
                out_hbm.at[pl.ds(pbase, spg)],
                msems[ms],
            ).wait()

        def reduce_chunk(side, mb, mrow0):
            buf = bufs[side]

            def cbody(c, carry):
                col = pl.ds(c * _LANES, _LANES)
                for j in range(cp):
                    r = j * _M
                    v = [buf[r + m, col] for m in range(_M)]
                    t8 = [v[m] + v[m + 8] for m in range(8)]
                    t4 = [t8[m] + t8[m + 4] for m in range(4)]
                    t2 = [t4[m] + t4[m + 2] for m in range(2)]
                    ssum = t2[0] + t2[1]
                    mb[mrow0 + j, col] = ssum * jnp.float32(1.0 / _M)
                return carry
            lax.fori_loop(0, nlc, cbody, 0)

        start_gather(0, 0)
        start_gather(1, 1)

        def run_sg(sg, ms):
            def gbody(g2, carry):
                ci0 = sg * 2 * gpsg + 2 * g2
                wait_gather(0)
                reduce_chunk(0, mbufs[ms], g2 * 2 * cp)
                @pl.when(ci0 + 2 < ntot)
                def _():
                    start_gather(ci0 + 2, 0)
                wait_gather(1)
                reduce_chunk(1, mbufs[ms], g2 * 2 * cp + cp)
                @pl.when(ci0 + 3 < ntot)
                def _():
                    start_gather(ci0 + 3, 1)
                return carry
            lax.fori_loop(0, gpsg, gbody, 0)
            pltpu.async_copy(
                mbufs[ms],
                out_hbm.at[pl.ds(pbase + sg * spg, spg)],
                msems[ms],
            )

        def sgbody(sg, carry):
            @pl.when(lax.rem(sg, 2) == 0)
            def _():
                @pl.when(sg >= 2)
                def _():
                    wait_mstore(0)
                run_sg(sg, 0)
            @pl.when(lax.rem(sg, 2) == 1)
            def _():
                @pl.when(sg >= 2)
                def _():
                    wait_mstore(1)
                run_sg(sg, 1)
            return carry

        lax.fori_loop(0, nsg, sgbody, 0)
        wait_mstore(0)
        wait_mstore(1)

    return k(table, idx)


def _repack_members(pi3):
    def kfn(i_ref, o_ref):
        arr = i_ref[...]
        for p in range(_P):
            o_ref[:, p * _M:(p + 1) * _M] = arr[:, p, :]

    return pl.pallas_call(
        kfn,
        out_shape=jax.ShapeDtypeStruct((_G, _P * _M), jnp.int32),
    )(pi3)


def _proto_argmin(rows2, emb, cm):
    b = emb.shape[0]
    bb = 64

    def kfn(r_ref, e_ref, cm_ref, bd_ref, bm_ref):
        r = r_ref[...]
        e = e_ref[...]
        dd = r - e[:, None, None, :]
        s = jnp.sqrt(jnp.sum(dd * dd, axis=-1) + 1e-12)
        best = jnp.min(s, axis=-1)
        i8 = lax.broadcasted_iota(jnp.int32, s.shape, 2)
        bi = jnp.min(jnp.where(s == best[..., None], i8, _P), axis=-1)
        cmv4 = cm_ref[...].reshape(bb, _TOPK, _P, _M)
        sel = i8[..., None] == bi[..., None, None]
        bm = jnp.sum(jnp.where(sel, cmv4, 0), axis=2)
        bd_ref[...] = best
        bm_ref[...] = bm

    return pl.pallas_call(
        kfn,
        grid=(b // bb,),
        in_specs=[
            pl.BlockSpec((bb, _TOPK, _P, _D), lambda i: (i, 0, 0, 0)),
            pl.BlockSpec((bb, _D), lambda i: (i, 0)),
            pl.BlockSpec((bb * _TOPK, _P * _M), lambda i: (i, 0)),
        ],
        out_specs=[
            pl.BlockSpec((bb, _TOPK), lambda i: (i, 0)),
            pl.BlockSpec((bb, _TOPK, _M), lambda i: (i, 0, 0)),
        ],
        out_shape=[
            jax.ShapeDtypeStruct((b, _TOPK), jnp.float32),
            jax.ShapeDtypeStruct((b, _TOPK, _M), jnp.int32),
        ],
    )(rows2, emb, cm)


def _member_argmin(rows3, emb, bm):
    b = emb.shape[0]
    bb = 64

    def kfn(r_ref, e_ref, bm_ref, bg_ref):
        r = r_ref[...]
        e = e_ref[...]
        dd = r - e[:, None, None, :]
        s = jnp.sqrt(jnp.sum(dd * dd, axis=-1) + 1e-12)
        best = jnp.min(s, axis=-1)
        i16 = lax.broadcasted_iota(jnp.int32, s.shape, 2)
        bi = jnp.min(jnp.where(s == best[..., None], i16, _M), axis=-1)
        bmv = bm_ref[...]
        bg = jnp.sum(jnp.where(i16 == bi[..., None], bmv, 0), axis=-1)
        bg_ref[...] = bg

    return pl.pallas_call(
        kfn,
        grid=(b // bb,),
        in_specs=[
            pl.BlockSpec((bb, _TOPK, _M, _D), lambda i: (i, 0, 0, 0)),
            pl.BlockSpec((bb, _D), lambda i: (i, 0)),
            pl.BlockSpec((bb, _TOPK, _M), lambda i: (i, 0, 0)),
        ],
        out_specs=pl.BlockSpec((bb, _TOPK), lambda i: (i, 0)),
        out_shape=jax.ShapeDtypeStruct((b, _TOPK), jnp.int32),
    )(rows3, emb, bm)


def _finish(bd, cp5, cand5, crows, ipreds, temp):
    b = bd.shape[0]

    def kfn(bd_ref, cp_ref, cd_ref, cr_ref, ip_ref, t_ref,
            olat_ref, olng_ref, ogc_ref):
        cr = cr_ref[...].reshape(b, _TOPK, 16)
        la_ref = None
        lo_ref = None
        la2 = cr[:, :, 0]
        lo2 = cr[:, :, 1]
        scores = -bd_ref[...]
        t = t_ref[0, 0]
        ex = jnp.exp(scores / t)
        probs = ex / jnp.sum(ex, axis=-1, keepdims=True)
        fp = cp_ref[...] * probs
        i5 = lax.broadcasted_iota(jnp.int32, fp.shape, 1)
        mx = jnp.max(fp, axis=-1)
        bi = jnp.min(jnp.where(fp == mx[:, None], i5, _TOPK), axis=-1)
        la = la2
        lo = lo2
        selb = i5 == bi[:, None]
        rlat = jnp.sum(jnp.where(selb, la, jnp.float32(0)), axis=-1)
        rlng = jnp.sum(jnp.where(selb, lo, jnp.float32(0)), axis=-1)
        deg = jnp.float32(math.pi / 180.0)
        lat1 = ip_ref[:, 0] * deg
        lng1 = ip_ref[:, 1] * deg
        lat2 = rlat * deg
        lng2 = rlng * deg
        sdlat = jnp.sin((lat2 - lat1) * 0.5)
        sdlng = jnp.sin((lng2 - lng1) * 0.5)
        a = sdlat * sdlat + jnp.cos(lat1) * jnp.cos(lat2) * sdlng * sdlng
        fi = jnp.where(a > jnp.float32(_ATHR), 0, bi)
        cd = cd_ref[...]

        self_ = i5 == fi[:, None]
        flat = jnp.sum(jnp.where(self_, la, jnp.float32(0)), axis=-1)
        flng = jnp.sum(jnp.where(self_, lo, jnp.float32(0)), axis=-1)
        gc = jnp.sum(jnp.where(self_, cd, 0), axis=-1)
        olat_ref[...] = flat
        olng_ref[...] = flng
        ogc_ref[...] = gc

    return pl.pallas_call(
        kfn,
        out_shape=[
            jax.ShapeDtypeStruct((b,), jnp.float32),
            jax.ShapeDtypeStruct((b,), jnp.float32),
            jax.ShapeDtypeStruct((b,), cand5.dtype),
        ],
    )(bd, cp5, cand5, crows, ipreds, temp)


def kernel(embedding, initial_preds, candidate_cells, candidate_probs,
           embeddings, proto_indices, dataset_latlng, temperature):
    b, d = embedding.shape
    fm = _repack_members(proto_indices.astype(jnp.int32))
    npad = ((_K0 + _K1) * _NS - _G * _P + max(_K0, _K1)) * _M
    idx1 = jnp.pad(fm.reshape(-1), (0, npad))
    pm = _gather_mean16(embeddings, idx1)
    cand = candidate_cells[:, :_TOPK].astype(jnp.int32)
    cand_flat = cand.reshape(-1)
    idx2 = (jnp.repeat(cand_flat, _P) * _P
            + jnp.tile(jnp.arange(_P, dtype=jnp.int32), b * _TOPK))
    rows2 = _gather_rows(pm, idx2).reshape(b, _TOPK, _P, d)
    cm = _gather_rows(fm, cand_flat)
    bd, bmsel = _proto_argmin(rows2, embedding, cm)
    rows3 = _gather_rows(embeddings, bmsel.reshape(-1))
    rows3 = rows3.reshape(b, _TOPK, _M, d)
    bg = _member_argmin(rows3, embedding, bmsel)
    llpad = jnp.pad(dataset_latlng, ((0, 0), (0, 14)))
    crows = _gather_rows(llpad, bg.reshape(-1))
    return _finish(bd, candidate_probs[:, :_TOPK], cand, crows,
                   initial_preds, jnp.reshape(temperature, (1, 1)))
